# Initial kernel scaffold; baseline (speedup 1.0000x reference)
import jax
import jax.numpy as jnp
from jax.experimental import pallas as pl


def kernel(inputs, edge_index, W1, W2):
    raise NotImplementedError("write your pallas kernel here")


# speedup vs baseline: 7.4731x; 7.4731x over previous
import functools

import jax
import jax.numpy as jnp
from jax import lax
from jax.experimental import pallas as pl
from jax.experimental.pallas import tpu as pltpu
from jax.experimental.pallas import tpu_sc as plsc

_NC = 2
_NS = 16
_NW = _NC * _NS
_CHUNK = 128


def _sc_aggregate(x_pad, src3, dst3, n_pad, d):
    nw, ch, _ = src3.shape
    rows_per_tile = n_pad // _NS
    copies_per_tile = rows_per_tile // _CHUNK
    mesh = plsc.VectorSubcoreMesh(core_axis_name="c", subcore_axis_name="s")

    @functools.partial(
        pl.kernel,
        out_type=jax.ShapeDtypeStruct((_NC, n_pad, d), jnp.float32),
        mesh=mesh,
        scratch_types=[
            pltpu.VMEM((ch, _CHUNK), jnp.int32),
            pltpu.VMEM((ch, _CHUNK), jnp.int32),
            pltpu.VMEM((_CHUNK, d), jnp.float32),
            pltpu.VMEM_SHARED((n_pad, d), jnp.float32),
            pltpu.SemaphoreType.DMA,
        ],
    )
    def agg_kernel(x_hbm, src_hbm, dst_hbm, out_hbm, src_v, dst_v, buf_v,
                   acc_sh, gsem):
        c = lax.axis_index("c")
        s = lax.axis_index("s")
        wid = s * _NC + c

        pltpu.sync_copy(src_hbm.at[wid], src_v)
        pltpu.sync_copy(dst_hbm.at[wid], dst_v)

        zeros16 = jnp.zeros((16,), jnp.float32)
        lanes = d // 16

        def zero_body(i, carry):
            r = i // lanes
            k = i % lanes
            buf_v[r, pl.ds(k * 16, 16)] = zeros16
            return carry

        lax.fori_loop(0, _CHUNK * lanes, zero_body, 0)
        row0 = s * rows_per_tile
        for r in range(copies_per_tile):
            pltpu.sync_copy(buf_v, acc_sh.at[pl.ds(row0 + r * _CHUNK,
                                                   _CHUNK)])
        plsc.subcore_barrier()

        def edge_body(j, carry):
            pltpu.async_copy(x_hbm.at[src_v.at[j]], buf_v, gsem).wait()
            pltpu.sync_copy(buf_v, acc_sh.at[dst_v.at[j]], add=True)
            return carry

        lax.fori_loop(0, ch, edge_body, 0)
        plsc.subcore_barrier()

        for r in range(copies_per_tile):
            row = row0 + r * _CHUNK
            pltpu.sync_copy(acc_sh.at[pl.ds(row, _CHUNK)], buf_v)
            pltpu.sync_copy(buf_v, out_hbm.at[c, pl.ds(row, _CHUNK)])

    return agg_kernel(x_pad, src3, dst3)


def _tc_project(parts, w, n_pad, d):
    br = 256
    grid = n_pad // br

    def body(p_ref, w_ref, o_ref):
        agg = p_ref[0] + p_ref[1]
        o_ref[...] = jnp.maximum(
            jnp.dot(agg, w_ref[...], preferred_element_type=jnp.float32), 0.0)

    return pl.pallas_call(
        body,
        grid=(grid,),
        in_specs=[
            pl.BlockSpec((2, br, d), lambda i: (0, i, 0)),
            pl.BlockSpec((d, d), lambda i: (0, 0)),
        ],
        out_specs=pl.BlockSpec((br, d), lambda i: (i, 0)),
        out_shape=jax.ShapeDtypeStruct((n_pad, d), jnp.float32),
    )(parts, w)


def kernel(inputs, edge_index, W1, W2):
    n, d = inputs.shape
    e = edge_index.shape[1]
    n_pad = ((n + _NS * _CHUNK - 1) // (_NS * _CHUNK)) * (_NS * _CHUNK)
    epw = -(-e // _NW)
    ch = -(-epw // _CHUNK)
    if ch % 2:
        ch += 1
    epw_pad = ch * _CHUNK

    x_pad = jnp.zeros((n_pad, d), jnp.float32).at[:n].set(inputs)

    pad_cols = _NW * epw_pad - e
    pad_idx = n + (jnp.arange(pad_cols, dtype=jnp.int32) % (n_pad - n))
    src = jnp.concatenate([edge_index[0], pad_idx]).reshape(_NW, ch, _CHUNK)
    dst = jnp.concatenate([edge_index[1], pad_idx]).reshape(_NW, ch, _CHUNK)

    p1 = _sc_aggregate(x_pad, src, dst, n_pad, d)
    h1 = _tc_project(p1, W1, n_pad, d)
    p2 = _sc_aggregate(h1, src, dst, n_pad, d)
    h2 = _tc_project(p2, W2, n_pad, d)

    return jnp.concatenate([inputs, h1[:n], h2[:n]], axis=0)


# speedup vs baseline: 10.5841x; 1.4163x over previous
import functools

import jax
import jax.numpy as jnp
from jax import lax
from jax.experimental import pallas as pl
from jax.experimental.pallas import tpu as pltpu
from jax.experimental.pallas import tpu_sc as plsc

_NC = 2
_NS = 16
_NW = _NC * _NS
_CHUNK = 128


def _sc_aggregate(x_pad, src3, dst3, n_pad, d):
    nw, ch, _ = src3.shape
    rows_per_tile = n_pad // _NS
    copies_per_tile = rows_per_tile // _CHUNK
    mesh = plsc.VectorSubcoreMesh(core_axis_name="c", subcore_axis_name="s")

    @functools.partial(
        pl.kernel,
        out_type=jax.ShapeDtypeStruct((_NC, n_pad, d), jnp.float32),
        mesh=mesh,
        scratch_types=[
            pltpu.VMEM((ch // 2, _CHUNK), jnp.int32),
            pltpu.VMEM((ch // 2, _CHUNK), jnp.int32),
            pltpu.VMEM((2, _CHUNK, d), jnp.float32),
            pltpu.VMEM_SHARED((n_pad, d), jnp.float32),
            pltpu.SemaphoreType.DMA,
            pltpu.SemaphoreType.DMA,
        ],
    )
    def agg_kernel(x_hbm, src_hbm, dst_hbm, out_hbm, src_v, dst_v, buf_v,
                   acc_sh, gsem0, gsem1):
        c = lax.axis_index("c")
        s = lax.axis_index("s")
        wid = s * _NC + c
        hch = ch // 2

        zeros16 = jnp.zeros((16,), jnp.float32)
        lanes = d // 16

        def zero_body(i, carry):
            r = i // lanes
            k = i % lanes
            buf_v[0, r, pl.ds(k * 16, 16)] = zeros16
            return carry

        lax.fori_loop(0, _CHUNK * lanes, zero_body, 0)
        row0 = s * rows_per_tile
        for r in range(copies_per_tile):
            pltpu.sync_copy(buf_v.at[0], acc_sh.at[pl.ds(row0 + r * _CHUNK,
                                                         _CHUNK)])
        plsc.subcore_barrier()

        for h in range(2):
            pltpu.sync_copy(src_hbm.at[wid, pl.ds(h * hch, hch)], src_v)
            pltpu.sync_copy(dst_hbm.at[wid, pl.ds(h * hch, hch)], dst_v)
            pltpu.async_copy(x_hbm.at[src_v.at[0]], buf_v.at[0], gsem0)
            pltpu.async_copy(x_hbm.at[src_v.at[1]], buf_v.at[1], gsem1)

            def pair_body(i, carry):
                j0 = 2 * i
                pltpu.make_async_copy(x_hbm.at[src_v.at[j0]], buf_v.at[0],
                                      gsem0).wait()
                pltpu.sync_copy(buf_v.at[0], acc_sh.at[dst_v.at[j0]],
                                add=True)
                pltpu.async_copy(x_hbm.at[src_v.at[j0 + 2]], buf_v.at[0],
                                 gsem0)
                pltpu.make_async_copy(x_hbm.at[src_v.at[j0 + 1]],
                                      buf_v.at[1], gsem1).wait()
                pltpu.sync_copy(buf_v.at[1], acc_sh.at[dst_v.at[j0 + 1]],
                                add=True)
                pltpu.async_copy(x_hbm.at[src_v.at[j0 + 3]], buf_v.at[1],
                                 gsem1)
                return carry

            lax.fori_loop(0, hch // 2 - 1, pair_body, 0)
            j0 = hch - 2
            pltpu.make_async_copy(x_hbm.at[src_v.at[j0]], buf_v.at[0],
                                  gsem0).wait()
            pltpu.sync_copy(buf_v.at[0], acc_sh.at[dst_v.at[j0]], add=True)
            pltpu.make_async_copy(x_hbm.at[src_v.at[j0 + 1]], buf_v.at[1],
                                  gsem1).wait()
            pltpu.sync_copy(buf_v.at[1], acc_sh.at[dst_v.at[j0 + 1]],
                            add=True)
        plsc.subcore_barrier()

        for r in range(copies_per_tile):
            row = row0 + r * _CHUNK
            pltpu.sync_copy(acc_sh.at[pl.ds(row, _CHUNK)], buf_v.at[0])
            pltpu.sync_copy(buf_v.at[0], out_hbm.at[c, pl.ds(row, _CHUNK)])

    return agg_kernel(x_pad, src3, dst3)


def _tc_project(parts, w, n_pad, d):
    br = 256
    grid = n_pad // br

    def body(p_ref, w_ref, o_ref):
        agg = p_ref[0] + p_ref[1]
        o_ref[...] = jnp.maximum(
            jnp.dot(agg, w_ref[...], preferred_element_type=jnp.float32), 0.0)

    return pl.pallas_call(
        body,
        grid=(grid,),
        in_specs=[
            pl.BlockSpec((2, br, d), lambda i: (0, i, 0)),
            pl.BlockSpec((d, d), lambda i: (0, 0)),
        ],
        out_specs=pl.BlockSpec((br, d), lambda i: (i, 0)),
        out_shape=jax.ShapeDtypeStruct((n_pad, d), jnp.float32),
    )(parts, w)


def kernel(inputs, edge_index, W1, W2):
    n, d = inputs.shape
    e = edge_index.shape[1]
    n_pad = ((n + _NS * _CHUNK - 1) // (_NS * _CHUNK)) * (_NS * _CHUNK)
    epw = -(-e // _NW)
    ch = ((-(-epw // _CHUNK) + 3) // 4) * 4
    epw_pad = ch * _CHUNK

    pad_cols = _NW * epw_pad - e
    spread = jnp.arange(pad_cols, dtype=jnp.int32)
    src = jnp.concatenate([edge_index[0], spread % n]).reshape(_NW, ch,
                                                              _CHUNK)
    dst = jnp.concatenate([edge_index[1], n + spread % (n_pad - n)
                           ]).reshape(_NW, ch, _CHUNK)

    p1 = _sc_aggregate(inputs, src, dst, n_pad, d)
    h1 = _tc_project(p1, W1, n_pad, d)
    p2 = _sc_aggregate(h1, src, dst, n_pad, d)
    h2 = _tc_project(p2, W2, n_pad, d)

    return jnp.concatenate([inputs, h1[:n], h2[:n]], axis=0)


# speedup vs baseline: 13.5911x; 1.2841x over previous
import functools

import jax
import jax.numpy as jnp
from jax import lax
from jax.experimental import pallas as pl
from jax.experimental.pallas import tpu as pltpu
from jax.experimental.pallas import tpu_sc as plsc

_NC = 2
_NS = 16
_NW = _NC * _NS
_CHUNK = 128


def _sc_aggregate(x, edges, n_pad, d):
    e = edges.shape[1]
    nfull = e // (_NW * _CHUNK)
    nextra = (e - _NW * nfull * _CHUNK) // _CHUNK
    ebase_extra = _NW * nfull * _CHUNK
    hch = nfull // 2
    rows_per_tile = n_pad // _NS
    copies_per_tile = rows_per_tile // _CHUNK
    mesh = plsc.VectorSubcoreMesh(core_axis_name="c", subcore_axis_name="s")

    @functools.partial(
        pl.kernel,
        out_type=jax.ShapeDtypeStruct((_NC, n_pad, d), jnp.float32),
        mesh=mesh,
        scratch_types=[
            pltpu.VMEM((2, hch * _CHUNK), jnp.int32),
            pltpu.VMEM((2, _CHUNK), jnp.int32),
            pltpu.VMEM((2, _CHUNK, d), jnp.float32),
            pltpu.VMEM_SHARED((n_pad, d), jnp.float32),
            pltpu.SemaphoreType.DMA,
            pltpu.SemaphoreType.DMA,
        ],
    )
    def agg_kernel(x_hbm, edge_hbm, out_hbm, idx_v, ext_v, buf_v, acc_sh,
                   gsem0, gsem1):
        c = lax.axis_index("c")
        s = lax.axis_index("s")
        wid = s * _NC + c
        ebase = wid * nfull * _CHUNK

        ext_wid = jnp.minimum(wid, nextra - 1)
        ext_cp = pltpu.async_copy(
            edge_hbm.at[:, pl.ds(ebase_extra + ext_wid * _CHUNK, _CHUNK)],
            ext_v, gsem1)

        zeros16 = jnp.zeros((16,), jnp.float32)
        lanes = d // 16

        def zero_body(r, carry):
            for k in range(lanes):
                buf_v[0, r, pl.ds(k * 16, 16)] = zeros16
            return carry

        lax.fori_loop(0, _CHUNK, zero_body, 0)
        row0 = s * rows_per_tile
        for r in range(copies_per_tile):
            pltpu.sync_copy(buf_v.at[0], acc_sh.at[pl.ds(row0 + r * _CHUNK,
                                                         _CHUNK)])
        ext_cp.wait()
        plsc.subcore_barrier()

        def sidx(j):
            return idx_v.at[0, pl.ds(j * _CHUNK, _CHUNK)]

        def didx(j):
            return idx_v.at[1, pl.ds(j * _CHUNK, _CHUNK)]

        for h in range(2):
            pltpu.sync_copy(
                edge_hbm.at[:, pl.ds(ebase + h * hch * _CHUNK,
                                     hch * _CHUNK)], idx_v)
            pltpu.async_copy(x_hbm.at[sidx(0)], buf_v.at[0], gsem0)
            pltpu.async_copy(x_hbm.at[sidx(1)], buf_v.at[1], gsem1)

            def pair_body(i, carry):
                j0 = 2 * i
                pltpu.make_async_copy(x_hbm.at[sidx(j0)], buf_v.at[0],
                                      gsem0).wait()
                pltpu.sync_copy(buf_v.at[0], acc_sh.at[didx(j0)], add=True)
                pltpu.async_copy(x_hbm.at[sidx(j0 + 2)], buf_v.at[0], gsem0)
                pltpu.make_async_copy(x_hbm.at[sidx(j0 + 1)], buf_v.at[1],
                                      gsem1).wait()
                pltpu.sync_copy(buf_v.at[1], acc_sh.at[didx(j0 + 1)],
                                add=True)

                @pl.when(j0 + 3 < hch)
                def _():
                    pltpu.async_copy(x_hbm.at[sidx(j0 + 3)], buf_v.at[1],
                                     gsem1)

                return carry

            lax.fori_loop(0, hch // 2 - 1, pair_body, 0)
            j0 = 2 * (hch // 2 - 1)
            pltpu.make_async_copy(x_hbm.at[sidx(j0)], buf_v.at[0],
                                  gsem0).wait()
            pltpu.sync_copy(buf_v.at[0], acc_sh.at[didx(j0)], add=True)
            pltpu.make_async_copy(x_hbm.at[sidx(j0 + 1)], buf_v.at[1],
                                  gsem1).wait()
            pltpu.sync_copy(buf_v.at[1], acc_sh.at[didx(j0 + 1)], add=True)

            if hch % 2:
                j1 = hch - 1
                pltpu.async_copy(x_hbm.at[sidx(j1)], buf_v.at[0],
                                 gsem0).wait()
                pltpu.sync_copy(buf_v.at[0], acc_sh.at[didx(j1)], add=True)

        @pl.when(wid < nextra)
        def _():
            pltpu.async_copy(x_hbm.at[ext_v.at[0]], buf_v.at[0],
                             gsem0).wait()
            pltpu.sync_copy(buf_v.at[0], acc_sh.at[ext_v.at[1]], add=True)

        plsc.subcore_barrier()

        pltpu.sync_copy(acc_sh.at[pl.ds(row0, rows_per_tile)],
                        out_hbm.at[c, pl.ds(row0, rows_per_tile)])

    return agg_kernel(x, edges)


def _tc_project(parts, w, n, d):
    br = 5000
    grid = n // br

    def body(p_ref, w_ref, o_ref):
        agg = p_ref[0] + p_ref[1]
        o_ref[...] = jnp.maximum(
            jnp.dot(agg, w_ref[...], preferred_element_type=jnp.float32), 0.0)

    return pl.pallas_call(
        body,
        grid=(grid,),
        in_specs=[
            pl.BlockSpec((2, br, d), lambda i: (0, i, 0)),
            pl.BlockSpec((d, d), lambda i: (0, 0)),
        ],
        out_specs=pl.BlockSpec((br, d), lambda i: (i, 0)),
        out_shape=jax.ShapeDtypeStruct((n, d), jnp.float32),
    )(parts, w)


def _tc_project_assemble(x, h1, parts, w, n, d):
    br = 5000
    sec = n // br

    def body(x_ref, h1_ref, p_ref, w_ref, o_ref):
        i = pl.program_id(0)

        @pl.when(i < sec)
        def _():
            o_ref[...] = x_ref[...]

        @pl.when((i >= sec) & (i < 2 * sec))
        def _():
            o_ref[...] = h1_ref[...]

        @pl.when(i >= 2 * sec)
        def _():
            agg = p_ref[0] + p_ref[1]
            o_ref[...] = jnp.maximum(
                jnp.dot(agg, w_ref[...],
                        preferred_element_type=jnp.float32), 0.0)

    clamp = lambda lo, hi: (lambda i: (jnp.clip(i - lo, 0, hi), 0))
    return pl.pallas_call(
        body,
        grid=(3 * sec,),
        in_specs=[
            pl.BlockSpec((br, d), clamp(0, sec - 1)),
            pl.BlockSpec((br, d), clamp(sec, sec - 1)),
            pl.BlockSpec((2, br, d),
                         lambda i: (0, jnp.clip(i - 2 * sec, 0, sec - 1), 0)),
            pl.BlockSpec((d, d), lambda i: (0, 0)),
        ],
        out_specs=pl.BlockSpec((br, d), lambda i: (i, 0)),
        out_shape=jax.ShapeDtypeStruct((3 * n, d), jnp.float32),
    )(x, h1, parts, w)


def kernel(inputs, edge_index, W1, W2):
    n, d = inputs.shape
    n_pad = ((n + _NS * _CHUNK - 1) // (_NS * _CHUNK)) * (_NS * _CHUNK)

    p1 = _sc_aggregate(inputs, edge_index, n_pad, d)
    h1 = _tc_project(p1, W1, n, d)
    p2 = _sc_aggregate(h1, edge_index, n_pad, d)
    return _tc_project_assemble(inputs, h1, p2, W2, n, d)


# speedup vs baseline: 14.0165x; 1.0313x over previous
import functools

import jax
import jax.numpy as jnp
from jax import lax
from jax.experimental import pallas as pl
from jax.experimental.pallas import tpu as pltpu
from jax.experimental.pallas import tpu_sc as plsc

_NC = 2
_NS = 16
_NW = _NC * _NS
_CHUNK = 128


def _sc_aggregate(x, edges, n_pad, d):
    e = edges.shape[1]
    nfull = e // (_NW * _CHUNK)
    nextra = (e - _NW * nfull * _CHUNK) // _CHUNK
    ebase_extra = _NW * nfull * _CHUNK
    hch = nfull // 2
    rows_per_tile = n_pad // _NS
    copies_per_tile = rows_per_tile // _CHUNK
    mesh = plsc.VectorSubcoreMesh(core_axis_name="c", subcore_axis_name="s")

    @functools.partial(
        pl.kernel,
        out_type=jax.ShapeDtypeStruct((_NC, n_pad, d), jnp.float32),
        mesh=mesh,
        scratch_types=[
            pltpu.VMEM((2, hch * _CHUNK), jnp.int32),
            pltpu.VMEM((2, _CHUNK), jnp.int32),
            pltpu.VMEM((2, _CHUNK, d), jnp.float32),
            pltpu.VMEM_SHARED((n_pad, d), jnp.float32),
            pltpu.SemaphoreType.DMA,
            pltpu.SemaphoreType.DMA,
        ],
    )
    def agg_kernel(x_hbm, edge_hbm, out_hbm, idx_v, ext_v, buf_v, acc_sh,
                   gsem0, gsem1):
        c = lax.axis_index("c")
        s = lax.axis_index("s")
        wid = s * _NC + c
        ebase = wid * nfull * _CHUNK

        ext_wid = jnp.minimum(wid, nextra - 1)
        ext_cp = pltpu.async_copy(
            edge_hbm.at[:, pl.ds(ebase_extra + ext_wid * _CHUNK, _CHUNK)],
            ext_v, gsem1)

        zeros16 = jnp.zeros((16,), jnp.float32)
        lanes = d // 16

        def zero_body(r, carry):
            for k in range(lanes):
                buf_v[0, r, pl.ds(k * 16, 16)] = zeros16
            return carry

        lax.fori_loop(0, _CHUNK, zero_body, 0)
        row0 = s * rows_per_tile
        for r in range(copies_per_tile):
            pltpu.sync_copy(buf_v.at[0], acc_sh.at[pl.ds(row0 + r * _CHUNK,
                                                         _CHUNK)])
        ext_cp.wait()
        plsc.subcore_barrier()

        def sidx(j):
            return idx_v.at[0, pl.ds(j * _CHUNK, _CHUNK)]

        def didx(j):
            return idx_v.at[1, pl.ds(j * _CHUNK, _CHUNK)]

        for h in range(2):
            pltpu.sync_copy(
                edge_hbm.at[:, pl.ds(ebase + h * hch * _CHUNK,
                                     hch * _CHUNK)], idx_v)
            pltpu.async_copy(x_hbm.at[sidx(0)], buf_v.at[0], gsem0)
            pltpu.async_copy(x_hbm.at[sidx(1)], buf_v.at[1], gsem1)

            def pair_body(i, carry):
                j0 = 2 * i
                pltpu.make_async_copy(x_hbm.at[sidx(j0)], buf_v.at[0],
                                      gsem0).wait()
                pltpu.sync_copy(buf_v.at[0], acc_sh.at[didx(j0)], add=True)
                pltpu.async_copy(x_hbm.at[sidx(j0 + 2)], buf_v.at[0], gsem0)
                pltpu.make_async_copy(x_hbm.at[sidx(j0 + 1)], buf_v.at[1],
                                      gsem1).wait()
                pltpu.sync_copy(buf_v.at[1], acc_sh.at[didx(j0 + 1)],
                                add=True)

                @pl.when(j0 + 3 < hch)
                def _():
                    pltpu.async_copy(x_hbm.at[sidx(j0 + 3)], buf_v.at[1],
                                     gsem1)

                return carry

            lax.fori_loop(0, hch // 2 - 1, pair_body, 0)
            j0 = 2 * (hch // 2 - 1)
            pltpu.make_async_copy(x_hbm.at[sidx(j0)], buf_v.at[0],
                                  gsem0).wait()
            pltpu.sync_copy(buf_v.at[0], acc_sh.at[didx(j0)], add=True)
            pltpu.make_async_copy(x_hbm.at[sidx(j0 + 1)], buf_v.at[1],
                                  gsem1).wait()
            pltpu.sync_copy(buf_v.at[1], acc_sh.at[didx(j0 + 1)], add=True)

            if hch % 2:
                j1 = hch - 1
                pltpu.async_copy(x_hbm.at[sidx(j1)], buf_v.at[0],
                                 gsem0).wait()
                pltpu.sync_copy(buf_v.at[0], acc_sh.at[didx(j1)], add=True)

        @pl.when(wid < nextra)
        def _():
            pltpu.async_copy(x_hbm.at[ext_v.at[0]], buf_v.at[0],
                             gsem0).wait()
            pltpu.sync_copy(buf_v.at[0], acc_sh.at[ext_v.at[1]], add=True)

        plsc.subcore_barrier()

        pltpu.sync_copy(acc_sh.at[pl.ds(row0, rows_per_tile)],
                        out_hbm.at[c, pl.ds(row0, rows_per_tile)])

    return agg_kernel(x, edges)


def _tc_project(parts, w, n, d):
    br = 5000
    grid = n // br

    def body(p_ref, w_ref, o_ref):
        agg = p_ref[0] + p_ref[1]
        o_ref[...] = jnp.maximum(
            jnp.dot(agg, w_ref[...], preferred_element_type=jnp.float32), 0.0)

    return pl.pallas_call(
        body,
        grid=(grid,),
        in_specs=[
            pl.BlockSpec((2, br, d), lambda i: (0, i, 0)),
            pl.BlockSpec((d, d), lambda i: (0, 0)),
        ],
        out_specs=pl.BlockSpec((br, d), lambda i: (i, 0)),
        out_shape=jax.ShapeDtypeStruct((n, d), jnp.float32),
    )(parts, w)


def _tc_copy_xh(x, h1, n, d):
    br = 5000
    sec = n // br

    def body(x_ref, h1_ref, o_ref):
        i = pl.program_id(0)

        @pl.when(i < sec)
        def _():
            o_ref[...] = x_ref[...]

        @pl.when(i >= sec)
        def _():
            o_ref[...] = h1_ref[...]

    clamp = lambda lo, hi: (lambda i: (jnp.clip(i - lo, 0, hi), 0))
    return pl.pallas_call(
        body,
        grid=(2 * sec,),
        in_specs=[
            pl.BlockSpec((br, d), clamp(0, sec - 1)),
            pl.BlockSpec((br, d), clamp(sec, sec - 1)),
        ],
        out_specs=pl.BlockSpec((br, d), lambda i: (i, 0)),
        out_shape=jax.ShapeDtypeStruct((3 * n, d), jnp.float32),
    )(x, h1)


def _tc_project_h2(out_xh, parts, w, n, d):
    br = 5000
    sec = n // br

    def body(o_in_ref, p_ref, w_ref, o_ref):
        agg = p_ref[0] + p_ref[1]
        o_ref[...] = jnp.maximum(
            jnp.dot(agg, w_ref[...], preferred_element_type=jnp.float32),
            0.0)

    return pl.pallas_call(
        body,
        grid=(sec,),
        in_specs=[
            pl.BlockSpec((8, d), lambda i: (0, 0)),
            pl.BlockSpec((2, br, d), lambda i: (0, i, 0)),
            pl.BlockSpec((d, d), lambda i: (0, 0)),
        ],
        out_specs=pl.BlockSpec((br, d), lambda i: (i + 2 * sec, 0)),
        out_shape=jax.ShapeDtypeStruct((3 * n, d), jnp.float32),
        input_output_aliases={0: 0},
    )(out_xh, parts, w)


def kernel(inputs, edge_index, W1, W2):
    n, d = inputs.shape
    n_pad = ((n + _NS * _CHUNK - 1) // (_NS * _CHUNK)) * (_NS * _CHUNK)

    p1 = _sc_aggregate(inputs, edge_index, n_pad, d)
    h1 = _tc_project(p1, W1, n, d)
    p2 = _sc_aggregate(h1, edge_index, n_pad, d)
    out_xh = _tc_copy_xh(inputs, h1, n, d)
    return _tc_project_h2(out_xh, p2, W2, n, d)
